# baseline (speedup 1.0000x reference)
import functools

import jax
import jax.numpy as jnp
from jax import lax
from jax.experimental import pallas as pl
from jax.experimental.pallas import tpu as pltpu
from jax.experimental.pallas import tpu_sc as plsc

N = 10000
E = 320000
D = 128
N_PAD = 10240
NC = 2
NS = 16
L = 16
NW = NC * NS
EPW = E // NW
K = 80
CH = EPW // K
EPW_P = EPW
STRIPE = N_PAD // NS
ZR = 128

_mesh = plsc.VectorSubcoreMesh(core_axis_name="c", subcore_axis_name="s")
_f32 = jnp.float32
_i32 = jnp.int32


def _dense_fwd_body(x_ref, w_ref, a_ref, wh_ref, st_ref):
    xb = x_ref[...]
    wh = jnp.dot(xb, w_ref[...], preferred_element_type=_f32)
    wh_ref[...] = wh
    st_ref[...] = lax.dot_general(a_ref[...], wh, (((1,), (1,)), ((), ())),
                                  preferred_element_type=_f32)


def _dense_fwd(x, W, A2p):
    B = 2048
    return pl.pallas_call(
        _dense_fwd_body,
        grid=(pl.cdiv(N, B),),
        in_specs=[pl.BlockSpec((B, D), lambda i: (i, 0)),
                  pl.BlockSpec((D, D), lambda i: (0, 0)),
                  pl.BlockSpec((8, D), lambda i: (0, 0))],
        out_specs=[pl.BlockSpec((B, D), lambda i: (i, 0)),
                   pl.BlockSpec((8, B), lambda i: (0, i))],
        out_shape=[jax.ShapeDtypeStruct((N, D), _f32),
                   jax.ShapeDtypeStruct((8, N), _f32)],
    )(x, W, A2p)


@functools.partial(
    pl.kernel, mesh=_mesh,
    compiler_params=pltpu.CompilerParams(needs_layout_passes=False),
    out_type=[jax.ShapeDtypeStruct((NW * EPW_P,), _f32),
              jax.ShapeDtypeStruct((NW * L,), _f32)],
    scratch_types=[pltpu.VMEM((N,), _f32),
                   pltpu.VMEM((N,), _f32),
                   pltpu.VMEM((EPW,), _i32),
                   pltpu.VMEM((EPW,), _i32),
                   pltpu.VMEM((EPW,), _f32),
                   pltpu.VMEM((L,), _f32),
                   pltpu.SemaphoreType.DMA])
def _edge_logits(st_hbm, src_hbm, dst_hbm, e_hbm, pmax_hbm,
                 ssrc_v, sdst_v, src_v, dst_v, e_v, max_v, sem):
    c = lax.axis_index("c")
    s = lax.axis_index("s")
    wid = s * NC + c
    base = wid * EPW
    pltpu.async_copy(st_hbm.at[0], ssrc_v, sem)
    pltpu.async_copy(st_hbm.at[1], sdst_v, sem)
    pltpu.async_copy(src_hbm.at[pl.ds(base, EPW)], src_v, sem)
    pltpu.async_copy(dst_hbm.at[pl.ds(base, EPW)], dst_v, sem)
    pltpu.make_async_copy(st_hbm.at[0], ssrc_v, sem).wait()
    pltpu.make_async_copy(st_hbm.at[1], sdst_v, sem).wait()
    pltpu.make_async_copy(src_hbm.at[pl.ds(0, EPW)], src_v, sem).wait()
    pltpu.make_async_copy(dst_hbm.at[pl.ds(0, EPW)], dst_v, sem).wait()
    max_v[...] = jnp.full((L,), -jnp.inf, _f32)

    def step(jq, carry):
        for q in range(5):
            j = jq * 5 + q
            sv = src_v[pl.ds(j * L, L)]
            dv = dst_v[pl.ds(j * L, L)]
            e = plsc.load_gather(ssrc_v, [sv]) + plsc.load_gather(sdst_v, [dv])
            e = jnp.where(e >= 0, e, 0.2 * e)
            e_v[pl.ds(j * L, L)] = e
            max_v[...] = jnp.maximum(max_v[...], e)
        return carry

    lax.fori_loop(0, EPW // L // 5, step, 0)
    pltpu.sync_copy(e_v, e_hbm.at[pl.ds(wid * EPW_P, EPW)])
    pltpu.sync_copy(max_v, pmax_hbm.at[pl.ds(wid * L, L)])


@functools.partial(
    pl.kernel, mesh=_mesh,
    compiler_params=pltpu.CompilerParams(needs_layout_passes=False),
    out_type=[jax.ShapeDtypeStruct((NC, N_PAD, D), _f32),
              jax.ShapeDtypeStruct((NC, N_PAD), _f32)],
    scratch_types=[pltpu.VMEM_SHARED((N_PAD, D), _f32),
                   pltpu.VMEM_SHARED((N_PAD,), _f32),
                   pltpu.VMEM((CH, K), _i32),
                   pltpu.VMEM((K,), _i32),
                   pltpu.VMEM((K,), _i32),
                   pltpu.VMEM((K,), _i32),
                   pltpu.VMEM((K,), _f32),
                   pltpu.VMEM((K,), _f32),
                   pltpu.VMEM((K,), _f32),
                   pltpu.VMEM((K,), _f32),
                   pltpu.VMEM((K,), _f32),
                   pltpu.VMEM((K,), _f32),
                   pltpu.VMEM((K, D), _f32),
                   pltpu.VMEM((K, D), _f32),
                   pltpu.VMEM((K, D), _f32),
                   pltpu.SemaphoreType.DMA,
                   pltpu.SemaphoreType.DMA,
                   pltpu.SemaphoreType.DMA,
                   pltpu.SemaphoreType.DMA,
                   pltpu.SemaphoreType.DMA,
                   pltpu.SemaphoreType.DMA,
                   pltpu.SemaphoreType.DMA,
                   pltpu.SemaphoreType.DMA,
                   pltpu.SemaphoreType.DMA])
def _edge_agg(src_hbm, dst_hbm, e_hbm, pmax_hbm, wh_hbm, u_hbm, den_hbm,
              u_sh, den_sh, dst_v, srcb0, srcb1, srcb2,
              eb0, eb1, eb2, pb0, pb1, pb2, rows0, rows1, rows2,
              semi0, semi1, semi2, semg0, semg1, semg2, sems0, sems1, sems2):
    c = lax.axis_index("c")
    s = lax.axis_index("s")
    wid = s * NC + c
    zero16 = jnp.zeros((L,), _f32)
    rows = (rows0, rows1, rows2)
    srcb = (srcb0, srcb1, srcb2)
    eb = (eb0, eb1, eb2)
    pb = (pb0, pb1, pb2)
    semi = (semi0, semi1, semi2)
    semg = (semg0, semg1, semg2)
    sems = (sems0, sems1, sems2)

    pltpu.async_copy(dst_hbm.at[wid], dst_v, semi2)

    for r in range(NW * L // D):
        pltpu.async_copy(pmax_hbm.at[pl.ds(r * D, D)], rows0.at[r], semg0)
    for r in range(NW * L // D):
        pltpu.make_async_copy(pmax_hbm.at[pl.ds(0, D)], rows0.at[r],
                              semg0).wait()
    macc = jnp.full((L,), -jnp.inf, _f32)
    for r in range(NW * L // D):
        for g in range(D // L):
            macc = jnp.maximum(macc, rows0[r, pl.ds(g * L, L)])
    m = jnp.max(macc)

    def zrow(i, carry):
        for d in range(D // L):
            rows0[i, pl.ds(d * L, L)] = zero16
        return carry
    lax.fori_loop(0, K, zrow, 0)

    for jj in range(K // L):
        pb0[pl.ds(jj * L, L)] = zero16

    stripe = s * STRIPE
    for r in range(STRIPE // K):
        pltpu.async_copy(rows0, u_sh.at[pl.ds(stripe + r * K, K)], semg1)
        pltpu.async_copy(pb0, den_sh.at[pl.ds(stripe + r * K, K)], semg1)
    for r in range(STRIPE // K):
        pltpu.make_async_copy(rows0, u_sh.at[pl.ds(stripe, K)], semg1).wait()
        pltpu.make_async_copy(pb0, den_sh.at[pl.ds(stripe, K)], semg1).wait()
    plsc.subcore_barrier()
    pltpu.make_async_copy(dst_hbm.at[wid], dst_v, semi2).wait()

    def fire_idx(t, b):
        base = wid * EPW_P + t * K
        pltpu.async_copy(src_hbm.at[pl.ds(base, K)], srcb[b], semi[b])
        pltpu.async_copy(e_hbm.at[pl.ds(base, K)], eb[b], semi[b])

    def wait_idx(b):
        pltpu.make_async_copy(src_hbm.at[pl.ds(0, K)], srcb[b],
                              semi[b]).wait()
        pltpu.make_async_copy(e_hbm.at[pl.ds(0, K)], eb[b],
                              semi[b]).wait()

    def fire_gather(b):
        pltpu.async_copy(wh_hbm.at[srcb[b]], rows[b], semg[b])

    def wait_gather(b):
        pltpu.make_async_copy(wh_hbm.at[pl.ds(0, K)], rows[b], semg[b]).wait()

    def compute_p(b):
        for jj in range(K // L):
            pb[b][pl.ds(jj * L, L)] = jnp.exp(eb[b][pl.ds(jj * L, L)] - m)

    def scale(b):
        @plsc.parallel_loop(0, K // L, 1, unroll=5)
        def grp(jj):
            jb = jj * L
            for j16 in range(L):
                j = jb + j16
                psc = plsc.load_gather(pb[b], [jnp.full((L,), j, _i32)])
                for d in range(D // L):
                    rows[b][j, pl.ds(d * L, L)] = (
                        rows[b][j, pl.ds(d * L, L)] * psc)

    def fire_scatter(t, b):
        pltpu.async_copy(rows[b], u_sh.at[dst_v.at[t]], sems[b], add=True)
        pltpu.async_copy(pb[b], den_sh.at[dst_v.at[t]], sems[b], add=True)

    def wait_scat(b):
        pltpu.make_async_copy(rows[b], u_sh.at[dst_v.at[0]], sems[b]).wait()
        pltpu.make_async_copy(pb[b], den_sh.at[dst_v.at[0]], sems[b]).wait()

    def body(t, b, first=False, fire_g=True, fire_i=True):
        bn = (b + 1) % 3
        b2 = (b + 2) % 3
        if fire_g:
            wait_idx(bn)
            if not first:
                wait_scat(bn)
            fire_gather(bn)
        wait_gather(b)
        compute_p(b)
        if fire_i:
            fire_idx(t + 2, b2)
        scale(b)
        fire_scatter(t, b)

    fire_idx(0, 0)
    wait_idx(0)
    fire_idx(1, 1)
    fire_gather(0)
    body(0, 0, first=True)
    body(1, 1, first=True)

    def group(g, carry):
        t0 = g * 3 + 2
        body(t0, 2)
        body(t0 + 1, 0)
        body(t0 + 2, 1)
        return carry

    lax.fori_loop(0, (CH - 5) // 3, group, 0)
    body(CH - 3, 2)
    body(CH - 2, 0, fire_i=False)
    body(CH - 1, 1, fire_g=False, fire_i=False)
    wait_scat(0)
    wait_scat(1)
    plsc.subcore_barrier()

    for r in range(STRIPE // K):
        pltpu.async_copy(u_sh.at[pl.ds(stripe + r * K, K)],
                         u_hbm.at[c, pl.ds(stripe + r * K, K)], semg0)
    pltpu.async_copy(den_sh.at[pl.ds(stripe, STRIPE)],
                     den_hbm.at[c, pl.ds(stripe, STRIPE)], semg0)
    for r in range(STRIPE // K):
        pltpu.make_async_copy(u_sh.at[pl.ds(stripe, K)],
                              u_hbm.at[c, pl.ds(stripe, K)], semg0).wait()
    pltpu.make_async_copy(den_sh.at[pl.ds(stripe, STRIPE)],
                          den_hbm.at[c, pl.ds(stripe, STRIPE)], semg0).wait()


def _finalize_body(u_ref, den_ref, out_ref):
    usum = u_ref[0] + u_ref[1]
    dsum = den_ref[0] + den_ref[1] + 1e-16
    v = usum / dsum[:, None]
    out_ref[...] = jnp.where(v > 0, v, jnp.exp(jnp.minimum(v, 0.0)) - 1.0)


def _finalize(u, den):
    B = 2048
    return pl.pallas_call(
        _finalize_body,
        grid=(pl.cdiv(N, B),),
        in_specs=[pl.BlockSpec((NC, B, D), lambda i: (0, i, 0)),
                  pl.BlockSpec((NC, B), lambda i: (0, i))],
        out_specs=pl.BlockSpec((B, D), lambda i: (i, 0)),
        out_shape=jax.ShapeDtypeStruct((N, D), _f32),
    )(u, den)


@jax.jit
def kernel(x, edge_index, W, a_src, a_dst):
    src = edge_index[0].astype(_i32)
    dst = edge_index[1].astype(_i32)
    A2p = jnp.concatenate(
        [a_src.T, a_dst.T, jnp.zeros((6, D), _f32)], axis=0)
    wh, st = _dense_fwd(x, W, A2p)
    e, pmax = _edge_logits(st, src, dst)
    dst3 = dst.reshape(NW, CH, K)
    u, den = _edge_agg(src, dst3, e, pmax, wh)
    return _finalize(u, den)


# speedup vs baseline: 1.4054x; 1.4054x over previous
import functools

import jax
import jax.numpy as jnp
from jax import lax
from jax.experimental import pallas as pl
from jax.experimental.pallas import tpu as pltpu
from jax.experimental.pallas import tpu_sc as plsc

N = 10000
E = 320000
D = 128
N_PAD = 10240
NC = 2
NS = 16
L = 16
NW = NC * NS
EPW = E // NW
K = 80
CH = EPW // K
EPW_P = EPW
STRIPE = N_PAD // NS
ZR = 128

_mesh = plsc.VectorSubcoreMesh(core_axis_name="c", subcore_axis_name="s")
_f32 = jnp.float32
_i32 = jnp.int32


def _dense_fwd_body(x_ref, w_ref, a_ref, wh_ref, st_ref):
    xb = x_ref[...]
    wh = jnp.dot(xb, w_ref[...], preferred_element_type=_f32)
    wh_ref[...] = wh
    st_ref[...] = lax.dot_general(a_ref[...], wh, (((1,), (1,)), ((), ())),
                                  preferred_element_type=_f32)


def _dense_fwd(x, W, A2p):
    B = 2048
    return pl.pallas_call(
        _dense_fwd_body,
        grid=(pl.cdiv(N, B),),
        in_specs=[pl.BlockSpec((B, D), lambda i: (i, 0)),
                  pl.BlockSpec((D, D), lambda i: (0, 0)),
                  pl.BlockSpec((8, D), lambda i: (0, 0))],
        out_specs=[pl.BlockSpec((B, D), lambda i: (i, 0)),
                   pl.BlockSpec((8, B), lambda i: (0, i))],
        out_shape=[jax.ShapeDtypeStruct((N, D), _f32),
                   jax.ShapeDtypeStruct((8, N), _f32)],
    )(x, W, A2p)


@functools.partial(
    pl.kernel, mesh=_mesh,
    compiler_params=pltpu.CompilerParams(needs_layout_passes=False),
    out_type=[jax.ShapeDtypeStruct((NW * EPW_P,), _f32),
              jax.ShapeDtypeStruct((NW * L,), _f32)],
    scratch_types=[pltpu.VMEM((N,), _f32),
                   pltpu.VMEM((N,), _f32),
                   pltpu.VMEM((EPW,), _i32),
                   pltpu.VMEM((EPW,), _i32),
                   pltpu.VMEM((EPW,), _f32),
                   pltpu.VMEM((L,), _f32),
                   pltpu.SemaphoreType.DMA])
def _edge_logits(st_hbm, src_hbm, dst_hbm, e_hbm, pmax_hbm,
                 ssrc_v, sdst_v, src_v, dst_v, e_v, max_v, sem):
    c = lax.axis_index("c")
    s = lax.axis_index("s")
    wid = s * NC + c
    base = wid * EPW
    pltpu.async_copy(st_hbm.at[0], ssrc_v, sem)
    pltpu.async_copy(st_hbm.at[1], sdst_v, sem)
    pltpu.async_copy(src_hbm.at[pl.ds(base, EPW)], src_v, sem)
    pltpu.async_copy(dst_hbm.at[pl.ds(base, EPW)], dst_v, sem)
    pltpu.make_async_copy(st_hbm.at[0], ssrc_v, sem).wait()
    pltpu.make_async_copy(st_hbm.at[1], sdst_v, sem).wait()
    pltpu.make_async_copy(src_hbm.at[pl.ds(0, EPW)], src_v, sem).wait()
    pltpu.make_async_copy(dst_hbm.at[pl.ds(0, EPW)], dst_v, sem).wait()
    max_v[...] = jnp.full((L,), -jnp.inf, _f32)

    def step(jq, carry):
        for q in range(5):
            j = jq * 5 + q
            sv = src_v[pl.ds(j * L, L)]
            dv = dst_v[pl.ds(j * L, L)]
            e = plsc.load_gather(ssrc_v, [sv]) + plsc.load_gather(sdst_v, [dv])
            e = jnp.where(e >= 0, e, 0.2 * e)
            e_v[pl.ds(j * L, L)] = e
            max_v[...] = jnp.maximum(max_v[...], e)
        return carry

    lax.fori_loop(0, EPW // L // 5, step, 0)
    pltpu.sync_copy(e_v, e_hbm.at[pl.ds(wid * EPW_P, EPW)])
    pltpu.sync_copy(max_v, pmax_hbm.at[pl.ds(wid * L, L)])


@functools.partial(
    pl.kernel, mesh=_mesh,
    compiler_params=pltpu.CompilerParams(needs_layout_passes=False),
    out_type=[jax.ShapeDtypeStruct((NC, N_PAD, D), _f32),
              jax.ShapeDtypeStruct((NC, N_PAD), _f32)],
    scratch_types=[pltpu.VMEM_SHARED((N_PAD, D), _f32),
                   pltpu.VMEM_SHARED((N_PAD,), _f32),
                   pltpu.VMEM((CH, K), _i32),
                   pltpu.VMEM((K,), _i32),
                   pltpu.VMEM((K,), _i32),
                   pltpu.VMEM((K,), _i32),
                   pltpu.VMEM((K,), _f32),
                   pltpu.VMEM((K,), _f32),
                   pltpu.VMEM((K,), _f32),
                   pltpu.VMEM((K,), _f32),
                   pltpu.VMEM((K,), _f32),
                   pltpu.VMEM((K,), _f32),
                   pltpu.VMEM((K, D), _f32),
                   pltpu.VMEM((K, D), _f32),
                   pltpu.VMEM((K, D), _f32),
                   pltpu.SemaphoreType.DMA,
                   pltpu.SemaphoreType.DMA,
                   pltpu.SemaphoreType.DMA,
                   pltpu.SemaphoreType.DMA,
                   pltpu.SemaphoreType.DMA,
                   pltpu.SemaphoreType.DMA,
                   pltpu.SemaphoreType.DMA,
                   pltpu.SemaphoreType.DMA,
                   pltpu.SemaphoreType.DMA])
def _edge_agg(src_hbm, dst_hbm, e_hbm, pmax_hbm, wh_hbm, u_hbm, den_hbm,
              u_sh, den_sh, dst_v, srcb0, srcb1, srcb2,
              eb0, eb1, eb2, pb0, pb1, pb2, rows0, rows1, rows2,
              semi0, semi1, semi2, semg0, semg1, semg2, sems0, sems1, sems2):
    c = lax.axis_index("c")
    s = lax.axis_index("s")
    wid = s * NC + c
    zero16 = jnp.zeros((L,), _f32)
    rows = (rows0, rows1, rows2)
    srcb = (srcb0, srcb1, srcb2)
    eb = (eb0, eb1, eb2)
    pb = (pb0, pb1, pb2)
    semi = (semi0, semi1, semi2)
    semg = (semg0, semg1, semg2)
    sems = (sems0, sems1, sems2)

    pltpu.async_copy(dst_hbm.at[wid], dst_v, semi2)

    for r in range(NW * L // D):
        pltpu.async_copy(pmax_hbm.at[pl.ds(r * D, D)], rows0.at[r], semg0)
    for r in range(NW * L // D):
        pltpu.make_async_copy(pmax_hbm.at[pl.ds(0, D)], rows0.at[r],
                              semg0).wait()
    macc = jnp.full((L,), -jnp.inf, _f32)
    for r in range(NW * L // D):
        for g in range(D // L):
            macc = jnp.maximum(macc, rows0[r, pl.ds(g * L, L)])
    m = jnp.max(macc)

    def zrow(i, carry):
        for d in range(D // L):
            rows0[i, pl.ds(d * L, L)] = zero16
        return carry
    lax.fori_loop(0, K, zrow, 0)

    for jj in range(K // L):
        pb0[pl.ds(jj * L, L)] = zero16

    stripe = s * STRIPE
    for r in range(STRIPE // K):
        pltpu.async_copy(rows0, u_sh.at[pl.ds(stripe + r * K, K)], semg1)
        pltpu.async_copy(pb0, den_sh.at[pl.ds(stripe + r * K, K)], semg1)
    for r in range(STRIPE // K):
        pltpu.make_async_copy(rows0, u_sh.at[pl.ds(stripe, K)], semg1).wait()
        pltpu.make_async_copy(pb0, den_sh.at[pl.ds(stripe, K)], semg1).wait()
    plsc.subcore_barrier()
    pltpu.make_async_copy(dst_hbm.at[wid], dst_v, semi2).wait()

    def fire_idx(t, b):
        base = wid * EPW_P + t * K
        pltpu.async_copy(src_hbm.at[pl.ds(base, K)], srcb[b], semi[b])
        pltpu.async_copy(e_hbm.at[pl.ds(base, K)], eb[b], semi[b])

    def wait_idx(b):
        pltpu.make_async_copy(src_hbm.at[pl.ds(0, K)], srcb[b],
                              semi[b]).wait()
        pltpu.make_async_copy(e_hbm.at[pl.ds(0, K)], eb[b],
                              semi[b]).wait()

    def fire_gather(b):
        pltpu.async_copy(wh_hbm.at[srcb[b]], rows[b], semg[b])

    def wait_gather(b):
        pltpu.make_async_copy(wh_hbm.at[pl.ds(0, K)], rows[b], semg[b]).wait()

    def compute_p(b):
        for jj in range(K // L):
            pb[b][pl.ds(jj * L, L)] = jnp.exp(eb[b][pl.ds(jj * L, L)] - m)

    def scale(b):
        @plsc.parallel_loop(0, K // L, 1)
        def grp(jj):
            jb = jj * L
            for j16 in range(L):
                j = jb + j16
                psc = plsc.load_gather(pb[b], [jnp.full((L,), j, _i32)])
                for d in range(D // L):
                    rows[b][j, pl.ds(d * L, L)] = (
                        rows[b][j, pl.ds(d * L, L)] * psc)

    def fire_scatter(t, b):
        pltpu.async_copy(rows[b], u_sh.at[dst_v.at[t]], sems[b], add=True)
        pltpu.async_copy(pb[b], den_sh.at[dst_v.at[t]], sems[b], add=True)

    def wait_scat(b):
        pltpu.make_async_copy(rows[b], u_sh.at[dst_v.at[0]], sems[b]).wait()
        pltpu.make_async_copy(pb[b], den_sh.at[dst_v.at[0]], sems[b]).wait()

    def body(t, b, first=False, fire_g=True, fire_i=True):
        bn = (b + 1) % 3
        b2 = (b + 2) % 3
        if fire_g:
            wait_idx(bn)
            if not first:
                wait_scat(bn)
            fire_gather(bn)
        wait_gather(b)
        compute_p(b)
        if fire_i:
            fire_idx(t + 2, b2)
        scale(b)
        fire_scatter(t, b)

    fire_idx(0, 0)
    wait_idx(0)
    fire_idx(1, 1)
    fire_gather(0)
    body(0, 0, first=True)
    body(1, 1, first=True)

    def group(g, carry):
        t0 = g * 3 + 2
        body(t0, 2)
        body(t0 + 1, 0)
        body(t0 + 2, 1)
        return carry

    lax.fori_loop(0, (CH - 5) // 3, group, 0)
    body(CH - 3, 2)
    body(CH - 2, 0, fire_i=False)
    body(CH - 1, 1, fire_g=False, fire_i=False)
    wait_scat(0)
    wait_scat(1)
    plsc.subcore_barrier()

    for r in range(STRIPE // K):
        pltpu.async_copy(u_sh.at[pl.ds(stripe + r * K, K)],
                         u_hbm.at[c, pl.ds(stripe + r * K, K)], semg0)
    pltpu.async_copy(den_sh.at[pl.ds(stripe, STRIPE)],
                     den_hbm.at[c, pl.ds(stripe, STRIPE)], semg0)
    for r in range(STRIPE // K):
        pltpu.make_async_copy(u_sh.at[pl.ds(stripe, K)],
                              u_hbm.at[c, pl.ds(stripe, K)], semg0).wait()
    pltpu.make_async_copy(den_sh.at[pl.ds(stripe, STRIPE)],
                          den_hbm.at[c, pl.ds(stripe, STRIPE)], semg0).wait()


def _finalize_body(u_ref, den_ref, out_ref):
    usum = u_ref[0] + u_ref[1]
    dsum = den_ref[0] + den_ref[1] + 1e-16
    v = usum / dsum[:, None]
    out_ref[...] = jnp.where(v > 0, v, jnp.exp(jnp.minimum(v, 0.0)) - 1.0)


def _finalize(u, den):
    B = 2048
    return pl.pallas_call(
        _finalize_body,
        grid=(pl.cdiv(N, B),),
        in_specs=[pl.BlockSpec((NC, B, D), lambda i: (0, i, 0)),
                  pl.BlockSpec((NC, B), lambda i: (0, i))],
        out_specs=pl.BlockSpec((B, D), lambda i: (i, 0)),
        out_shape=jax.ShapeDtypeStruct((N, D), _f32),
    )(u, den)


@jax.jit
def kernel(x, edge_index, W, a_src, a_dst):
    src = edge_index[0].astype(_i32)
    dst = edge_index[1].astype(_i32)
    A2p = jnp.concatenate(
        [a_src.T, a_dst.T, jnp.zeros((6, D), _f32)], axis=0)
    wh, st = _dense_fwd(x, W, A2p)
    e, pmax = _edge_logits(st, src, dst)
    dst3 = dst.reshape(NW, CH, K)
    u, den = _edge_agg(src, dst3, e, pmax, wh)
    return _finalize(u, den)


# speedup vs baseline: 1.4678x; 1.0444x over previous
import functools

import jax
import jax.numpy as jnp
from jax import lax
from jax.experimental import pallas as pl
from jax.experimental.pallas import tpu as pltpu
from jax.experimental.pallas import tpu_sc as plsc

N = 10000
E = 320000
D = 128
N_PAD = 10240
NC = 2
NS = 16
L = 16
NW = NC * NS
EPW = E // NW
K = 80
CH = EPW // K
EPW_P = EPW
STRIPE = N_PAD // NS
ZR = 128

_mesh = plsc.VectorSubcoreMesh(core_axis_name="c", subcore_axis_name="s")
_f32 = jnp.float32
_i32 = jnp.int32


def _dense_fwd_body(x_ref, w_ref, a_ref, wh_ref, st_ref):
    xb = x_ref[...]
    wh = jnp.dot(xb, w_ref[...], preferred_element_type=_f32)
    wh_ref[...] = wh
    st_ref[...] = lax.dot_general(a_ref[...], wh, (((1,), (1,)), ((), ())),
                                  preferred_element_type=_f32)


def _dense_fwd(x, W, A2p):
    B = 2048
    return pl.pallas_call(
        _dense_fwd_body,
        grid=(pl.cdiv(N, B),),
        in_specs=[pl.BlockSpec((B, D), lambda i: (i, 0)),
                  pl.BlockSpec((D, D), lambda i: (0, 0)),
                  pl.BlockSpec((8, D), lambda i: (0, 0))],
        out_specs=[pl.BlockSpec((B, D), lambda i: (i, 0)),
                   pl.BlockSpec((8, B), lambda i: (0, i))],
        out_shape=[jax.ShapeDtypeStruct((N, D), _f32),
                   jax.ShapeDtypeStruct((8, N), _f32)],
    )(x, W, A2p)


@functools.partial(
    pl.kernel, mesh=_mesh,
    compiler_params=pltpu.CompilerParams(needs_layout_passes=False),
    out_type=[jax.ShapeDtypeStruct((NW * EPW_P,), _f32),
              jax.ShapeDtypeStruct((NW * L,), _f32)],
    scratch_types=[pltpu.VMEM((N,), _f32),
                   pltpu.VMEM((N,), _f32),
                   pltpu.VMEM((EPW,), _i32),
                   pltpu.VMEM((EPW,), _i32),
                   pltpu.VMEM((EPW,), _f32),
                   pltpu.VMEM((L,), _f32),
                   pltpu.SemaphoreType.DMA])
def _edge_logits(st_hbm, src_hbm, dst_hbm, e_hbm, pmax_hbm,
                 ssrc_v, sdst_v, src_v, dst_v, e_v, max_v, sem):
    c = lax.axis_index("c")
    s = lax.axis_index("s")
    wid = s * NC + c
    base = wid * EPW
    pltpu.async_copy(st_hbm.at[0], ssrc_v, sem)
    pltpu.async_copy(st_hbm.at[1], sdst_v, sem)
    pltpu.async_copy(src_hbm.at[pl.ds(base, EPW)], src_v, sem)
    pltpu.async_copy(dst_hbm.at[pl.ds(base, EPW)], dst_v, sem)
    pltpu.make_async_copy(st_hbm.at[0], ssrc_v, sem).wait()
    pltpu.make_async_copy(st_hbm.at[1], sdst_v, sem).wait()
    pltpu.make_async_copy(src_hbm.at[pl.ds(0, EPW)], src_v, sem).wait()
    pltpu.make_async_copy(dst_hbm.at[pl.ds(0, EPW)], dst_v, sem).wait()
    @plsc.parallel_loop(0, EPW // L // 5, 1,
                        carry=jnp.full((L,), -jnp.inf, _f32))
    def step(jq, mc):
        for q in range(5):
            j = jq * 5 + q
            sv = src_v[pl.ds(j * L, L)]
            dv = dst_v[pl.ds(j * L, L)]
            e = plsc.load_gather(ssrc_v, [sv]) + plsc.load_gather(sdst_v, [dv])
            e = jnp.where(e >= 0, e, 0.2 * e)
            e_v[pl.ds(j * L, L)] = e
            mc = jnp.maximum(mc, e)
        return mc

    max_v[...] = step
    pltpu.sync_copy(e_v, e_hbm.at[pl.ds(wid * EPW_P, EPW)])
    pltpu.sync_copy(max_v, pmax_hbm.at[pl.ds(wid * L, L)])


@functools.partial(
    pl.kernel, mesh=_mesh,
    compiler_params=pltpu.CompilerParams(needs_layout_passes=False),
    out_type=[jax.ShapeDtypeStruct((NC, N_PAD, D), _f32),
              jax.ShapeDtypeStruct((NC, N_PAD), _f32)],
    scratch_types=[pltpu.VMEM_SHARED((N_PAD, D), _f32),
                   pltpu.VMEM_SHARED((N_PAD,), _f32),
                   pltpu.VMEM((CH, K), _i32),
                   pltpu.VMEM((K,), _i32),
                   pltpu.VMEM((K,), _i32),
                   pltpu.VMEM((K,), _i32),
                   pltpu.VMEM((K,), _f32),
                   pltpu.VMEM((K,), _f32),
                   pltpu.VMEM((K,), _f32),
                   pltpu.VMEM((K,), _f32),
                   pltpu.VMEM((K,), _f32),
                   pltpu.VMEM((K,), _f32),
                   pltpu.VMEM((K, D), _f32),
                   pltpu.VMEM((K, D), _f32),
                   pltpu.VMEM((K, D), _f32),
                   pltpu.SemaphoreType.DMA,
                   pltpu.SemaphoreType.DMA,
                   pltpu.SemaphoreType.DMA,
                   pltpu.SemaphoreType.DMA,
                   pltpu.SemaphoreType.DMA,
                   pltpu.SemaphoreType.DMA,
                   pltpu.SemaphoreType.DMA,
                   pltpu.SemaphoreType.DMA,
                   pltpu.SemaphoreType.DMA])
def _edge_agg(src_hbm, dst_hbm, e_hbm, pmax_hbm, wh_hbm, u_hbm, den_hbm,
              u_sh, den_sh, dst_v, srcb0, srcb1, srcb2,
              eb0, eb1, eb2, pb0, pb1, pb2, rows0, rows1, rows2,
              semi0, semi1, semi2, semg0, semg1, semg2, sems0, sems1, sems2):
    c = lax.axis_index("c")
    s = lax.axis_index("s")
    wid = s * NC + c
    zero16 = jnp.zeros((L,), _f32)
    rows = (rows0, rows1, rows2)
    srcb = (srcb0, srcb1, srcb2)
    eb = (eb0, eb1, eb2)
    pb = (pb0, pb1, pb2)
    semi = (semi0, semi1, semi2)
    semg = (semg0, semg1, semg2)
    sems = (sems0, sems1, sems2)

    pltpu.async_copy(dst_hbm.at[wid], dst_v, semi2)

    for r in range(NW * L // D):
        pltpu.async_copy(pmax_hbm.at[pl.ds(r * D, D)], rows0.at[r], semg0)
    for r in range(NW * L // D):
        pltpu.make_async_copy(pmax_hbm.at[pl.ds(0, D)], rows0.at[r],
                              semg0).wait()
    macc = jnp.full((L,), -jnp.inf, _f32)
    for r in range(NW * L // D):
        for g in range(D // L):
            macc = jnp.maximum(macc, rows0[r, pl.ds(g * L, L)])
    m = jnp.max(macc)

    def zrow(i, carry):
        for d in range(D // L):
            rows0[i, pl.ds(d * L, L)] = zero16
        return carry
    lax.fori_loop(0, K, zrow, 0)

    for jj in range(K // L):
        pb0[pl.ds(jj * L, L)] = zero16

    stripe = s * STRIPE
    for r in range(STRIPE // K):
        pltpu.async_copy(rows0, u_sh.at[pl.ds(stripe + r * K, K)], semg1)
        pltpu.async_copy(pb0, den_sh.at[pl.ds(stripe + r * K, K)], semg1)
    for r in range(STRIPE // K):
        pltpu.make_async_copy(rows0, u_sh.at[pl.ds(stripe, K)], semg1).wait()
        pltpu.make_async_copy(pb0, den_sh.at[pl.ds(stripe, K)], semg1).wait()
    plsc.subcore_barrier()
    pltpu.make_async_copy(dst_hbm.at[wid], dst_v, semi2).wait()

    def fire_idx(t, b):
        base = wid * EPW_P + t * K
        pltpu.async_copy(src_hbm.at[pl.ds(base, K)], srcb[b], semi[b])
        pltpu.async_copy(e_hbm.at[pl.ds(base, K)], eb[b], semi[b])

    def wait_idx(b):
        pltpu.make_async_copy(src_hbm.at[pl.ds(0, K)], srcb[b],
                              semi[b]).wait()
        pltpu.make_async_copy(e_hbm.at[pl.ds(0, K)], eb[b],
                              semi[b]).wait()

    def fire_gather(b):
        pltpu.async_copy(wh_hbm.at[srcb[b]], rows[b], semg[b])

    def wait_gather(b):
        pltpu.make_async_copy(wh_hbm.at[pl.ds(0, K)], rows[b], semg[b]).wait()

    def compute_p(b):
        for jj in range(K // L):
            pb[b][pl.ds(jj * L, L)] = jnp.exp(eb[b][pl.ds(jj * L, L)] - m)

    def scale(b):
        @plsc.parallel_loop(0, K // L, 1)
        def grp(jj):
            jb = jj * L
            for j16 in range(L):
                j = jb + j16
                psc = plsc.load_gather(pb[b], [jnp.full((L,), j, _i32)])
                for d in range(D // L):
                    rows[b][j, pl.ds(d * L, L)] = (
                        rows[b][j, pl.ds(d * L, L)] * psc)

    def fire_scatter(t, b):
        pltpu.async_copy(rows[b], u_sh.at[dst_v.at[t]], sems[b], add=True)
        pltpu.async_copy(pb[b], den_sh.at[dst_v.at[t]], sems[b], add=True)

    def wait_scat(b):
        pltpu.make_async_copy(rows[b], u_sh.at[dst_v.at[0]], sems[b]).wait()
        pltpu.make_async_copy(pb[b], den_sh.at[dst_v.at[0]], sems[b]).wait()

    def body(t, b, first=False, fire_g=True, fire_i=True):
        bn = (b + 1) % 3
        b2 = (b + 2) % 3
        if fire_g:
            wait_idx(bn)
            if not first:
                wait_scat(bn)
            fire_gather(bn)
        wait_gather(b)
        compute_p(b)
        if fire_i:
            fire_idx(t + 2, b2)
        scale(b)
        fire_scatter(t, b)

    fire_idx(0, 0)
    wait_idx(0)
    fire_idx(1, 1)
    fire_gather(0)
    body(0, 0, first=True)
    body(1, 1, first=True)

    def group(g, carry):
        t0 = g * 3 + 2
        body(t0, 2)
        body(t0 + 1, 0)
        body(t0 + 2, 1)
        return carry

    lax.fori_loop(0, (CH - 5) // 3, group, 0)
    body(CH - 3, 2)
    body(CH - 2, 0, fire_i=False)
    body(CH - 1, 1, fire_g=False, fire_i=False)
    wait_scat(0)
    wait_scat(1)
    plsc.subcore_barrier()

    for r in range(STRIPE // K):
        pltpu.async_copy(u_sh.at[pl.ds(stripe + r * K, K)],
                         u_hbm.at[c, pl.ds(stripe + r * K, K)], semg0)
    pltpu.async_copy(den_sh.at[pl.ds(stripe, STRIPE)],
                     den_hbm.at[c, pl.ds(stripe, STRIPE)], semg0)
    for r in range(STRIPE // K):
        pltpu.make_async_copy(u_sh.at[pl.ds(stripe, K)],
                              u_hbm.at[c, pl.ds(stripe, K)], semg0).wait()
    pltpu.make_async_copy(den_sh.at[pl.ds(stripe, STRIPE)],
                          den_hbm.at[c, pl.ds(stripe, STRIPE)], semg0).wait()


def _finalize_body(u_ref, den_ref, out_ref):
    usum = u_ref[0] + u_ref[1]
    dsum = den_ref[0] + den_ref[1] + 1e-16
    v = usum / dsum[:, None]
    out_ref[...] = jnp.where(v > 0, v, jnp.exp(jnp.minimum(v, 0.0)) - 1.0)


def _finalize(u, den):
    B = 2048
    return pl.pallas_call(
        _finalize_body,
        grid=(pl.cdiv(N, B),),
        in_specs=[pl.BlockSpec((NC, B, D), lambda i: (0, i, 0)),
                  pl.BlockSpec((NC, B), lambda i: (0, i))],
        out_specs=pl.BlockSpec((B, D), lambda i: (i, 0)),
        out_shape=jax.ShapeDtypeStruct((N, D), _f32),
    )(u, den)


@jax.jit
def kernel(x, edge_index, W, a_src, a_dst):
    src = edge_index[0].astype(_i32)
    dst = edge_index[1].astype(_i32)
    A2p = jnp.concatenate(
        [a_src.T, a_dst.T, jnp.zeros((6, D), _f32)], axis=0)
    wh, st = _dense_fwd(x, W, A2p)
    e, pmax = _edge_logits(st, src, dst)
    dst3 = dst.reshape(NW, CH, K)
    u, den = _edge_agg(src, dst3, e, pmax, wh)
    return _finalize(u, den)
